# baseline (speedup 1.0000x reference)
import functools

import jax
import jax.numpy as jnp
from jax import lax
from jax.experimental import pallas as pl
from jax.experimental.pallas import tpu as pltpu
from jax.experimental.pallas import tpu_sc as plsc

N = 4096
HSD = 2048
DENC = 1024
DT = 256
DR = 128
KT = 1024
KR = 512
CONF = 64
PH = 512

BN = 1024
GRID = N // BN

NC, NS = 2, 16
NW = NC * NS
RPW = N // NW

_NT = (((1,), (1,)), ((), ()))


def _full(shape):
    return pl.BlockSpec(shape, lambda *_: tuple(0 for _ in shape))


def _prep_body(mt_ref, lt_ref, wp_ref, wmu_ref, wlv_ref, mplv_ref, w2_ref):
    mt = mt_ref[...]
    mplv_ref[...] = jnp.concatenate([mt, lt_ref[...]], axis=1)
    wp = wp_ref[...]
    wmu = wmu_ref[...]
    wlv = wlv_ref[...]
    pl2 = jnp.sum(wp * wp) + jnp.sum(wmu * wmu) + jnp.sum(wlv * wlv)
    ca = jnp.sum(mt * mt)
    lane = lax.broadcasted_iota(jnp.int32, (1, 128), 1)
    w2_ref[...] = (jnp.where(lane == 0, pl2, 0.0)
                   + jnp.where(lane == 1, ca, 0.0))


def _prep(mu_table, logvar_table, Wp1, Wmu, Wlv):
    return pl.pallas_call(
        _prep_body,
        in_specs=[_full((KT, CONF)), _full((KT, CONF)),
                  _full((2 * DT, PH)), _full((PH, CONF)), _full((PH, CONF))],
        out_specs=[_full((KT, 2 * CONF)), _full((1, 128))],
        out_shape=[
            jax.ShapeDtypeStruct((KT, 2 * CONF), jnp.float32),
            jax.ShapeDtypeStruct((1, 128), jnp.float32),
        ],
    )(mu_table, logvar_table, Wp1, Wmu, Wlv)


def _enc_body(h_ref, we_ref, be_ref, wt_ref, bt_ref, wr_ref, br_ref,
              zt_ref, zr_ref, h1_s):
    i = pl.program_id(0)
    h1 = h1_s[pl.ds(((i + 1) % 2) * BN, BN), :]
    zt_ref[...] = jnp.dot(h1, wt_ref[...], preferred_element_type=jnp.float32) + bt_ref[...]
    zr_ref[...] = jnp.dot(h1, wr_ref[...], preferred_element_type=jnp.float32) + br_ref[...]
    h1_s[pl.ds((i % 2) * BN, BN), :] = jax.nn.gelu(
        jnp.dot(h_ref[...], we_ref[...], preferred_element_type=jnp.float32)
        + be_ref[...])


def _encoder(h, W_enc, b_enc, W_tr, b_tr, W_re, b_re):
    return pl.pallas_call(
        _enc_body,
        grid=(GRID + 1,),
        in_specs=[
            pl.BlockSpec((BN, HSD), lambda i: (jnp.minimum(i, GRID - 1), 0)),
            _full((HSD, DENC)), _full((1, DENC)),
            _full((DENC, DT)), _full((1, DT)),
            _full((DENC, DR)), _full((1, DR)),
        ],
        out_specs=[
            pl.BlockSpec((BN, DT), lambda i: (jnp.maximum(i - 1, 0), 0)),
            pl.BlockSpec((BN, DR), lambda i: (jnp.maximum(i - 1, 0), 0)),
        ],
        out_shape=[
            jax.ShapeDtypeStruct((N, DT), jnp.float32),
            jax.ShapeDtypeStruct((N, DR), jnp.float32),
        ],
        scratch_shapes=[pltpu.VMEM((2 * BN, DENC), jnp.float32)],
    )(h, W_enc, b_enc.reshape(1, -1), W_tr, b_tr.reshape(1, -1),
      W_re, b_re.reshape(1, -1))


def _vq_body(znt_ref, zsqt_ref, znr_ref, zsqr_ref,
             cbnt_ref, cn2t_ref, cbnr_ref, cn2r_ref,
             itr_ref, mse_ref, dt_s, dr_s, acc):
    i = pl.program_id(0)

    @pl.when(i == 0)
    def _init():
        acc[0] = 0.0
        acc[1] = 0.0

    off = ((i + 1) % 2) * BN
    dprev = dt_s[pl.ds(off, BN), :]
    itr_ref[0, 0, :] = jnp.argmin(dprev, axis=1).astype(jnp.int32)
    acc[0] += jnp.where(i > 0, jnp.sum(jnp.min(dprev, axis=1)), 0.0)
    drprev = dr_s[pl.ds(off, BN), :]
    acc[1] += jnp.where(i > 0, jnp.sum(jnp.min(drprev, axis=1)), 0.0)

    woff = (i % 2) * BN
    dt_s[pl.ds(woff, BN), :] = (
        (zsqt_ref[...] - 2.0 * lax.dot_general(
            znt_ref[...], cbnt_ref[...], _NT, preferred_element_type=jnp.float32))
        + cn2t_ref[...])
    dr_s[pl.ds(woff, BN), :] = (
        (zsqr_ref[...] - 2.0 * lax.dot_general(
            znr_ref[...], cbnr_ref[...], _NT, preferred_element_type=jnp.float32))
        + cn2r_ref[...])

    @pl.when(i == GRID)
    def _fin():
        lane = lax.broadcasted_iota(jnp.int32, (1, 128), 1)
        mse_ref[...] = (jnp.where(lane == 0, acc[0], 0.0)
                        + jnp.where(lane == 1, acc[1], 0.0))


def _vq(zn_tr, zsq_tr, zn_re, zsq_re, cbn_tr, cn2_tr, cbn_re, cn2_re):
    return pl.pallas_call(
        _vq_body,
        grid=(GRID + 1,),
        in_specs=[
            pl.BlockSpec((BN, DT), lambda i: (jnp.minimum(i, GRID - 1), 0)),
            pl.BlockSpec((BN, 1), lambda i: (jnp.minimum(i, GRID - 1), 0)),
            pl.BlockSpec((BN, DR), lambda i: (jnp.minimum(i, GRID - 1), 0)),
            pl.BlockSpec((BN, 1), lambda i: (jnp.minimum(i, GRID - 1), 0)),
            _full((KT, DT)), _full((1, KT)),
            _full((KR, DR)), _full((1, KR)),
        ],
        out_specs=[
            pl.BlockSpec((1, 1, BN), lambda i: (jnp.maximum(i - 1, 0), 0, 0)),
            pl.BlockSpec((1, 128), lambda i: (0, 0)),
        ],
        out_shape=[
            jax.ShapeDtypeStruct((GRID, 1, BN), jnp.int32),
            jax.ShapeDtypeStruct((1, 128), jnp.float32),
        ],
        scratch_shapes=[
            pltpu.VMEM((2 * BN, KT), jnp.float32),
            pltpu.VMEM((2 * BN, KR), jnp.float32),
            pltpu.SMEM((2,), jnp.float32),
        ],
    )(zn_tr, zsq_tr, zn_re, zsq_re, cbn_tr, cn2_tr, cbn_re, cn2_re)


def _sc_gather(cbn, mplv_table, idx):
    mesh = plsc.VectorSubcoreMesh(core_axis_name="c", subcore_axis_name="s")

    @functools.partial(
        pl.kernel, mesh=mesh,
        out_type=[jax.ShapeDtypeStruct((N, DT), jnp.float32),
                  jax.ShapeDtypeStruct((N, 2 * CONF), jnp.float32)],
        scratch_types=[pltpu.VMEM((RPW,), jnp.int32),
                       pltpu.VMEM((RPW, DT), jnp.float32),
                       pltpu.VMEM((RPW, 2 * CONF), jnp.float32),
                       pltpu.SemaphoreType.DMA,
                       pltpu.SemaphoreType.DMA],
    )
    def k(cbn_hbm, mplv_hbm, idx_hbm, q_out, mplv_out,
          idx_v, q_v, m_v, s1, s2):
        wid = lax.axis_index("s") * NC + lax.axis_index("c")
        base = wid * RPW
        pltpu.sync_copy(idx_hbm.at[pl.ds(base, RPW)], idx_v)
        c1 = pltpu.async_copy(cbn_hbm.at[idx_v], q_v, s1)
        c2 = pltpu.async_copy(mplv_hbm.at[idx_v], m_v, s2)
        c1.wait()
        c2.wait()
        pltpu.sync_copy(q_v, q_out.at[pl.ds(base, RPW)])
        pltpu.sync_copy(m_v, mplv_out.at[pl.ds(base, RPW)])

    return k(cbn, mplv_table, idx)


def _s3_body(z_ref, q_ref, mplv_ref, wp_ref, bp_ref, wmu_ref, bmu_ref,
             wlv_ref, blv_ref, u_ref, sc_ref, hid_s, acc):
    i = pl.program_id(0)

    @pl.when(i == 0)
    def _init():
        acc[0] = 0.0
        acc[1] = 0.0
        acc[2] = 0.0

    hid = hid_s[pl.ds(((i + 1) % 2) * BN, BN), :]
    mu = jnp.dot(hid, wmu_ref[...], preferred_element_type=jnp.float32) + bmu_ref[...]
    lv = jnp.dot(hid, wlv_ref[...], preferred_element_type=jnp.float32) + blv_ref[...]
    u_ref[...] = mu
    mplv = mplv_ref[...]
    mp = mplv[:, :CONF]
    lp = mplv[:, CONF:]
    kl_terms = lp - lv + (jnp.exp(lv) + (mu - mp) ** 2) / jnp.exp(lp) - 1.0
    acc[0] += jnp.where(i > 0, jnp.sum(kl_terms), 0.0)
    acc[1] += jnp.where(i > 0, jnp.sum(jnp.abs(mu)), 0.0)
    acc[2] += jnp.where(i > 0, jnp.sum(mp * mp), 0.0)

    xp = jnp.concatenate([z_ref[...], q_ref[...]], axis=1)
    hid_s[pl.ds((i % 2) * BN, BN), :] = jax.nn.gelu(
        jnp.dot(xp, wp_ref[...], preferred_element_type=jnp.float32) + bp_ref[...])

    @pl.when(i == GRID)
    def _fin():
        lane = lax.broadcasted_iota(jnp.int32, (1, 128), 1)
        sc_ref[...] = (jnp.where(lane == 0, acc[0], 0.0)
                       + jnp.where(lane == 1, acc[1], 0.0)
                       + jnp.where(lane == 2, acc[2], 0.0))


def _stage3(z_tr, q_tr, mplv_prior, Wp1, bp1, Wmu, bmu, Wlv, blv):
    return pl.pallas_call(
        _s3_body,
        grid=(GRID + 1,),
        in_specs=[
            pl.BlockSpec((BN, DT), lambda i: (jnp.minimum(i, GRID - 1), 0)),
            pl.BlockSpec((BN, DT), lambda i: (jnp.minimum(i, GRID - 1), 0)),
            pl.BlockSpec((BN, 2 * CONF), lambda i: (jnp.maximum(i - 1, 0), 0)),
            _full((2 * DT, PH)), _full((1, PH)),
            _full((PH, CONF)), _full((1, CONF)),
            _full((PH, CONF)), _full((1, CONF)),
        ],
        out_specs=[
            pl.BlockSpec((BN, CONF), lambda i: (jnp.maximum(i - 1, 0), 0)),
            pl.BlockSpec((1, 128), lambda i: (0, 0)),
        ],
        out_shape=[
            jax.ShapeDtypeStruct((N, CONF), jnp.float32),
            jax.ShapeDtypeStruct((1, 128), jnp.float32),
        ],
        scratch_shapes=[
            pltpu.VMEM((2 * BN, PH), jnp.float32),
            pltpu.SMEM((3,), jnp.float32),
        ],
    )(z_tr, q_tr, mplv_prior, Wp1, bp1.reshape(1, -1),
      Wmu, bmu.reshape(1, -1), Wlv, blv.reshape(1, -1))


def kernel(h, W_enc, b_enc, W_tr, b_tr, W_re, b_re, cb_tr, cb_re, mu_table,
           logvar_table, Wp1, bp1, Wmu, bmu, Wlv, blv, global_step, training):
    mplv_table, w2 = _prep(mu_table, logvar_table, Wp1, Wmu, Wlv)
    z_tr, z_re = _encoder(h, W_enc, b_enc, W_tr, b_tr, W_re, b_re)

    zn_tr = z_tr / (jnp.linalg.norm(z_tr, axis=-1, keepdims=True) + 1e-6)
    zn_re = z_re / (jnp.linalg.norm(z_re, axis=-1, keepdims=True) + 1e-6)
    cbn_tr = cb_tr / (jnp.linalg.norm(cb_tr, axis=-1, keepdims=True) + 1e-6)
    cbn_re = cb_re / (jnp.linalg.norm(cb_re, axis=-1, keepdims=True) + 1e-6)
    zsq_tr = jnp.sum(zn_tr * zn_tr, axis=-1, keepdims=True)
    zsq_re = jnp.sum(zn_re * zn_re, axis=-1, keepdims=True)
    cn2_tr = jnp.sum(cbn_tr * cbn_tr, axis=-1)[None, :]
    cn2_re = jnp.sum(cbn_re * cbn_re, axis=-1)[None, :]

    idx3_tr, mse = _vq(zn_tr, zsq_tr, zn_re, zsq_re,
                       cbn_tr, cn2_tr, cbn_re, cn2_re)
    idx_tr = idx3_tr.reshape(N)
    q_tr, mplv_prior = _sc_gather(cbn_tr, mplv_table, idx_tr)
    u_post, sc = _stage3(z_tr, q_tr, mplv_prior,
                         Wp1, bp1, Wmu, bmu, Wlv, blv)
    mse_tr = mse[0, 0] / (N * DT)
    mse_re = mse[0, 1] / (N * DR)
    quant = 1.25 * (mse_tr + mse_re)
    kl = 0.5 * sc[0, 0] / N
    sparsity = sc[0, 1] / (N * CONF)
    prior_reg = sc[0, 2] / (N * CONF)
    post_l2 = w2[0, 0]
    code_align = w2[0, 1] / (KT * CONF)
    conf_a = 0.1 * code_align + 0.01 * prior_reg
    conf_b = 0.1 * kl + 0.001 * post_l2 + 0.001 * sparsity
    conf = jnp.where(global_step % 3 == 0, conf_a, conf_b)
    loss = jnp.where(training, quant + conf, 0.0).astype(jnp.float32)
    return q_tr, u_post, loss


# speedup vs baseline: 1.0042x; 1.0042x over previous
import functools

import jax
import jax.numpy as jnp
from jax import lax
from jax.experimental import pallas as pl
from jax.experimental.pallas import tpu as pltpu
from jax.experimental.pallas import tpu_sc as plsc

N = 4096
HSD = 2048
DENC = 1024
DT = 256
DR = 128
KT = 1024
KR = 512
CONF = 64
PH = 512

BN = 512
GRID = N // BN

NC, NS = 2, 16
NW = NC * NS
RPW = N // NW

_NT = (((1,), (1,)), ((), ()))


def _full(shape):
    return pl.BlockSpec(shape, lambda *_: tuple(0 for _ in shape))


def _prep_body(mt_ref, lt_ref, wp_ref, wmu_ref, wlv_ref, mplv_ref, w2_ref):
    mt = mt_ref[...]
    mplv_ref[...] = jnp.concatenate([mt, lt_ref[...]], axis=1)
    wp = wp_ref[...]
    wmu = wmu_ref[...]
    wlv = wlv_ref[...]
    pl2 = jnp.sum(wp * wp) + jnp.sum(wmu * wmu) + jnp.sum(wlv * wlv)
    ca = jnp.sum(mt * mt)
    lane = lax.broadcasted_iota(jnp.int32, (1, 128), 1)
    w2_ref[...] = (jnp.where(lane == 0, pl2, 0.0)
                   + jnp.where(lane == 1, ca, 0.0))


def _prep(mu_table, logvar_table, Wp1, Wmu, Wlv):
    return pl.pallas_call(
        _prep_body,
        in_specs=[_full((KT, CONF)), _full((KT, CONF)),
                  _full((2 * DT, PH)), _full((PH, CONF)), _full((PH, CONF))],
        out_specs=[_full((KT, 2 * CONF)), _full((1, 128))],
        out_shape=[
            jax.ShapeDtypeStruct((KT, 2 * CONF), jnp.float32),
            jax.ShapeDtypeStruct((1, 128), jnp.float32),
        ],
    )(mu_table, logvar_table, Wp1, Wmu, Wlv)


def _enc_body(h_ref, we_ref, be_ref, wt_ref, bt_ref, wr_ref, br_ref,
              zt_ref, zr_ref, h1_s):
    i = pl.program_id(0)
    h1 = h1_s[pl.ds(((i + 1) % 2) * BN, BN), :]
    zt_ref[...] = jnp.dot(h1, wt_ref[...], preferred_element_type=jnp.float32) + bt_ref[...]
    zr_ref[...] = jnp.dot(h1, wr_ref[...], preferred_element_type=jnp.float32) + br_ref[...]
    h1_s[pl.ds((i % 2) * BN, BN), :] = jax.nn.gelu(
        jnp.dot(h_ref[...], we_ref[...], preferred_element_type=jnp.float32)
        + be_ref[...])


def _encoder(h, W_enc, b_enc, W_tr, b_tr, W_re, b_re):
    return pl.pallas_call(
        _enc_body,
        grid=(GRID + 1,),
        in_specs=[
            pl.BlockSpec((BN, HSD), lambda i: (jnp.minimum(i, GRID - 1), 0)),
            _full((HSD, DENC)), _full((1, DENC)),
            _full((DENC, DT)), _full((1, DT)),
            _full((DENC, DR)), _full((1, DR)),
        ],
        out_specs=[
            pl.BlockSpec((BN, DT), lambda i: (jnp.maximum(i - 1, 0), 0)),
            pl.BlockSpec((BN, DR), lambda i: (jnp.maximum(i - 1, 0), 0)),
        ],
        out_shape=[
            jax.ShapeDtypeStruct((N, DT), jnp.float32),
            jax.ShapeDtypeStruct((N, DR), jnp.float32),
        ],
        scratch_shapes=[pltpu.VMEM((2 * BN, DENC), jnp.float32)],
    )(h, W_enc, b_enc.reshape(1, -1), W_tr, b_tr.reshape(1, -1),
      W_re, b_re.reshape(1, -1))


def _vq_body(znt_ref, zsqt_ref, znr_ref, zsqr_ref,
             cbnt_ref, cn2t_ref, cbnr_ref, cn2r_ref,
             itr_ref, mse_ref, dt_s, dr_s, acc):
    i = pl.program_id(0)

    @pl.when(i == 0)
    def _init():
        acc[0] = 0.0
        acc[1] = 0.0

    off = ((i + 1) % 2) * BN
    dprev = dt_s[pl.ds(off, BN), :]
    itr_ref[0, 0, :] = jnp.argmin(dprev, axis=1).astype(jnp.int32)
    acc[0] += jnp.where(i > 0, jnp.sum(jnp.min(dprev, axis=1)), 0.0)
    drprev = dr_s[pl.ds(off, BN), :]
    acc[1] += jnp.where(i > 0, jnp.sum(jnp.min(drprev, axis=1)), 0.0)

    woff = (i % 2) * BN
    dt_s[pl.ds(woff, BN), :] = (
        (zsqt_ref[...] - 2.0 * lax.dot_general(
            znt_ref[...], cbnt_ref[...], _NT, preferred_element_type=jnp.float32))
        + cn2t_ref[...])
    dr_s[pl.ds(woff, BN), :] = (
        (zsqr_ref[...] - 2.0 * lax.dot_general(
            znr_ref[...], cbnr_ref[...], _NT, preferred_element_type=jnp.float32))
        + cn2r_ref[...])

    @pl.when(i == GRID)
    def _fin():
        lane = lax.broadcasted_iota(jnp.int32, (1, 128), 1)
        mse_ref[...] = (jnp.where(lane == 0, acc[0], 0.0)
                        + jnp.where(lane == 1, acc[1], 0.0))


def _vq(zn_tr, zsq_tr, zn_re, zsq_re, cbn_tr, cn2_tr, cbn_re, cn2_re):
    return pl.pallas_call(
        _vq_body,
        grid=(GRID + 1,),
        in_specs=[
            pl.BlockSpec((BN, DT), lambda i: (jnp.minimum(i, GRID - 1), 0)),
            pl.BlockSpec((BN, 1), lambda i: (jnp.minimum(i, GRID - 1), 0)),
            pl.BlockSpec((BN, DR), lambda i: (jnp.minimum(i, GRID - 1), 0)),
            pl.BlockSpec((BN, 1), lambda i: (jnp.minimum(i, GRID - 1), 0)),
            _full((KT, DT)), _full((1, KT)),
            _full((KR, DR)), _full((1, KR)),
        ],
        out_specs=[
            pl.BlockSpec((1, 1, BN), lambda i: (jnp.maximum(i - 1, 0), 0, 0)),
            pl.BlockSpec((1, 128), lambda i: (0, 0)),
        ],
        out_shape=[
            jax.ShapeDtypeStruct((GRID, 1, BN), jnp.int32),
            jax.ShapeDtypeStruct((1, 128), jnp.float32),
        ],
        scratch_shapes=[
            pltpu.VMEM((2 * BN, KT), jnp.float32),
            pltpu.VMEM((2 * BN, KR), jnp.float32),
            pltpu.SMEM((2,), jnp.float32),
        ],
    )(zn_tr, zsq_tr, zn_re, zsq_re, cbn_tr, cn2_tr, cbn_re, cn2_re)


def _sc_gather(cbn, mplv_table, idx):
    mesh = plsc.VectorSubcoreMesh(core_axis_name="c", subcore_axis_name="s")

    @functools.partial(
        pl.kernel, mesh=mesh,
        out_type=[jax.ShapeDtypeStruct((N, DT), jnp.float32),
                  jax.ShapeDtypeStruct((N, 2 * CONF), jnp.float32)],
        scratch_types=[pltpu.VMEM((RPW,), jnp.int32),
                       pltpu.VMEM((RPW, DT), jnp.float32),
                       pltpu.VMEM((RPW, 2 * CONF), jnp.float32),
                       pltpu.SemaphoreType.DMA,
                       pltpu.SemaphoreType.DMA],
    )
    def k(cbn_hbm, mplv_hbm, idx_hbm, q_out, mplv_out,
          idx_v, q_v, m_v, s1, s2):
        wid = lax.axis_index("s") * NC + lax.axis_index("c")
        base = wid * RPW
        pltpu.sync_copy(idx_hbm.at[pl.ds(base, RPW)], idx_v)
        c1 = pltpu.async_copy(cbn_hbm.at[idx_v], q_v, s1)
        c2 = pltpu.async_copy(mplv_hbm.at[idx_v], m_v, s2)
        c1.wait()
        c2.wait()
        pltpu.sync_copy(q_v, q_out.at[pl.ds(base, RPW)])
        pltpu.sync_copy(m_v, mplv_out.at[pl.ds(base, RPW)])

    return k(cbn, mplv_table, idx)


def _s3_body(z_ref, q_ref, mplv_ref, wp_ref, bp_ref, wmu_ref, bmu_ref,
             wlv_ref, blv_ref, u_ref, sc_ref, hid_s, acc):
    i = pl.program_id(0)

    @pl.when(i == 0)
    def _init():
        acc[0] = 0.0
        acc[1] = 0.0
        acc[2] = 0.0

    hid = hid_s[pl.ds(((i + 1) % 2) * BN, BN), :]
    mu = jnp.dot(hid, wmu_ref[...], preferred_element_type=jnp.float32) + bmu_ref[...]
    lv = jnp.dot(hid, wlv_ref[...], preferred_element_type=jnp.float32) + blv_ref[...]
    u_ref[...] = mu
    mplv = mplv_ref[...]
    mp = mplv[:, :CONF]
    lp = mplv[:, CONF:]
    kl_terms = lp - lv + (jnp.exp(lv) + (mu - mp) ** 2) / jnp.exp(lp) - 1.0
    acc[0] += jnp.where(i > 0, jnp.sum(kl_terms), 0.0)
    acc[1] += jnp.where(i > 0, jnp.sum(jnp.abs(mu)), 0.0)
    acc[2] += jnp.where(i > 0, jnp.sum(mp * mp), 0.0)

    xp = jnp.concatenate([z_ref[...], q_ref[...]], axis=1)
    hid_s[pl.ds((i % 2) * BN, BN), :] = jax.nn.gelu(
        jnp.dot(xp, wp_ref[...], preferred_element_type=jnp.float32) + bp_ref[...])

    @pl.when(i == GRID)
    def _fin():
        lane = lax.broadcasted_iota(jnp.int32, (1, 128), 1)
        sc_ref[...] = (jnp.where(lane == 0, acc[0], 0.0)
                       + jnp.where(lane == 1, acc[1], 0.0)
                       + jnp.where(lane == 2, acc[2], 0.0))


def _stage3(z_tr, q_tr, mplv_prior, Wp1, bp1, Wmu, bmu, Wlv, blv):
    return pl.pallas_call(
        _s3_body,
        grid=(GRID + 1,),
        in_specs=[
            pl.BlockSpec((BN, DT), lambda i: (jnp.minimum(i, GRID - 1), 0)),
            pl.BlockSpec((BN, DT), lambda i: (jnp.minimum(i, GRID - 1), 0)),
            pl.BlockSpec((BN, 2 * CONF), lambda i: (jnp.maximum(i - 1, 0), 0)),
            _full((2 * DT, PH)), _full((1, PH)),
            _full((PH, CONF)), _full((1, CONF)),
            _full((PH, CONF)), _full((1, CONF)),
        ],
        out_specs=[
            pl.BlockSpec((BN, CONF), lambda i: (jnp.maximum(i - 1, 0), 0)),
            pl.BlockSpec((1, 128), lambda i: (0, 0)),
        ],
        out_shape=[
            jax.ShapeDtypeStruct((N, CONF), jnp.float32),
            jax.ShapeDtypeStruct((1, 128), jnp.float32),
        ],
        scratch_shapes=[
            pltpu.VMEM((2 * BN, PH), jnp.float32),
            pltpu.SMEM((3,), jnp.float32),
        ],
    )(z_tr, q_tr, mplv_prior, Wp1, bp1.reshape(1, -1),
      Wmu, bmu.reshape(1, -1), Wlv, blv.reshape(1, -1))


def kernel(h, W_enc, b_enc, W_tr, b_tr, W_re, b_re, cb_tr, cb_re, mu_table,
           logvar_table, Wp1, bp1, Wmu, bmu, Wlv, blv, global_step, training):
    mplv_table, w2 = _prep(mu_table, logvar_table, Wp1, Wmu, Wlv)
    z_tr, z_re = _encoder(h, W_enc, b_enc, W_tr, b_tr, W_re, b_re)

    zn_tr = z_tr / (jnp.linalg.norm(z_tr, axis=-1, keepdims=True) + 1e-6)
    zn_re = z_re / (jnp.linalg.norm(z_re, axis=-1, keepdims=True) + 1e-6)
    cbn_tr = cb_tr / (jnp.linalg.norm(cb_tr, axis=-1, keepdims=True) + 1e-6)
    cbn_re = cb_re / (jnp.linalg.norm(cb_re, axis=-1, keepdims=True) + 1e-6)
    zsq_tr = jnp.sum(zn_tr * zn_tr, axis=-1, keepdims=True)
    zsq_re = jnp.sum(zn_re * zn_re, axis=-1, keepdims=True)
    cn2_tr = jnp.sum(cbn_tr * cbn_tr, axis=-1)[None, :]
    cn2_re = jnp.sum(cbn_re * cbn_re, axis=-1)[None, :]

    idx3_tr, mse = _vq(zn_tr, zsq_tr, zn_re, zsq_re,
                       cbn_tr, cn2_tr, cbn_re, cn2_re)
    idx_tr = idx3_tr.reshape(N)
    q_tr, mplv_prior = _sc_gather(cbn_tr, mplv_table, idx_tr)
    u_post, sc = _stage3(z_tr, q_tr, mplv_prior,
                         Wp1, bp1, Wmu, bmu, Wlv, blv)
    mse_tr = mse[0, 0] / (N * DT)
    mse_re = mse[0, 1] / (N * DR)
    quant = 1.25 * (mse_tr + mse_re)
    kl = 0.5 * sc[0, 0] / N
    sparsity = sc[0, 1] / (N * CONF)
    prior_reg = sc[0, 2] / (N * CONF)
    post_l2 = w2[0, 0]
    code_align = w2[0, 1] / (KT * CONF)
    conf_a = 0.1 * code_align + 0.01 * prior_reg
    conf_b = 0.1 * kl + 0.001 * post_l2 + 0.001 * sparsity
    conf = jnp.where(global_step % 3 == 0, conf_a, conf_b)
    loss = jnp.where(training, quant + conf, 0.0).astype(jnp.float32)
    return q_tr, u_post, loss


# speedup vs baseline: 1.0131x; 1.0088x over previous
import functools

import jax
import jax.numpy as jnp
from jax import lax
from jax.experimental import pallas as pl
from jax.experimental.pallas import tpu as pltpu
from jax.experimental.pallas import tpu_sc as plsc

N = 4096
HSD = 2048
DENC = 1024
DT = 256
DR = 128
KT = 1024
KR = 512
CONF = 64
PH = 512

BN = 512
GRID = N // BN

NC, NS = 2, 16
NW = NC * NS
RPW = N // NW

_NT = (((1,), (1,)), ((), ()))


def _full(shape):
    return pl.BlockSpec(shape, lambda *_: tuple(0 for _ in shape))


def _enc_body(h_ref, we_ref, be_ref, wt_ref, bt_ref, wr_ref, br_ref,
              zt_ref, zr_ref, h1_s):
    i = pl.program_id(0)
    h1 = h1_s[pl.ds(((i + 1) % 2) * BN, BN), :]
    zt_ref[...] = jnp.dot(h1, wt_ref[...], preferred_element_type=jnp.float32) + bt_ref[...]
    zr_ref[...] = jnp.dot(h1, wr_ref[...], preferred_element_type=jnp.float32) + br_ref[...]
    h1_s[pl.ds((i % 2) * BN, BN), :] = jax.nn.gelu(
        jnp.dot(h_ref[...], we_ref[...], preferred_element_type=jnp.float32)
        + be_ref[...])


def _encoder(h, W_enc, b_enc, W_tr, b_tr, W_re, b_re):
    return pl.pallas_call(
        _enc_body,
        grid=(GRID + 1,),
        in_specs=[
            pl.BlockSpec((BN, HSD), lambda i: (jnp.minimum(i, GRID - 1), 0)),
            _full((HSD, DENC)), _full((1, DENC)),
            _full((DENC, DT)), _full((1, DT)),
            _full((DENC, DR)), _full((1, DR)),
        ],
        out_specs=[
            pl.BlockSpec((BN, DT), lambda i: (jnp.maximum(i - 1, 0), 0)),
            pl.BlockSpec((BN, DR), lambda i: (jnp.maximum(i - 1, 0), 0)),
        ],
        out_shape=[
            jax.ShapeDtypeStruct((N, DT), jnp.float32),
            jax.ShapeDtypeStruct((N, DR), jnp.float32),
        ],
        scratch_shapes=[pltpu.VMEM((2 * BN, DENC), jnp.float32)],
    )(h, W_enc, b_enc.reshape(1, -1), W_tr, b_tr.reshape(1, -1),
      W_re, b_re.reshape(1, -1))


def _vq_body(znt_ref, zsqt_ref, znr_ref, zsqr_ref,
             cbnt_ref, cn2t_ref, cbnr_ref, cn2r_ref,
             mt_ref, wp_ref, wmu_ref, wlv_ref,
             itr_ref, mse_ref, dt_s, dr_s, acc):
    i = pl.program_id(0)

    @pl.when(i == 0)
    def _init():
        acc[0] = 0.0
        acc[1] = 0.0

    off = ((i + 1) % 2) * BN
    dprev = dt_s[pl.ds(off, BN), :]
    itr_ref[0, 0, :] = jnp.argmin(dprev, axis=1).astype(jnp.int32)
    acc[0] += jnp.where(i > 0, jnp.sum(jnp.min(dprev, axis=1)), 0.0)
    drprev = dr_s[pl.ds(off, BN), :]
    acc[1] += jnp.where(i > 0, jnp.sum(jnp.min(drprev, axis=1)), 0.0)

    woff = (i % 2) * BN
    dt_s[pl.ds(woff, BN), :] = (
        (zsqt_ref[...] - 2.0 * lax.dot_general(
            znt_ref[...], cbnt_ref[...], _NT, preferred_element_type=jnp.float32))
        + cn2t_ref[...])
    dr_s[pl.ds(woff, BN), :] = (
        (zsqr_ref[...] - 2.0 * lax.dot_general(
            znr_ref[...], cbnr_ref[...], _NT, preferred_element_type=jnp.float32))
        + cn2r_ref[...])

    @pl.when(i == GRID)
    def _fin():
        wp = wp_ref[...]
        wmu = wmu_ref[...]
        wlv = wlv_ref[...]
        pl2 = jnp.sum(wp * wp) + jnp.sum(wmu * wmu) + jnp.sum(wlv * wlv)
        mt = mt_ref[...]
        ca = jnp.sum(mt * mt)
        lane = lax.broadcasted_iota(jnp.int32, (1, 128), 1)
        mse_ref[...] = (jnp.where(lane == 0, acc[0], 0.0)
                        + jnp.where(lane == 1, acc[1], 0.0)
                        + jnp.where(lane == 2, pl2, 0.0)
                        + jnp.where(lane == 3, ca, 0.0))


def _vq(zn_tr, zsq_tr, zn_re, zsq_re, cbn_tr, cn2_tr, cbn_re, cn2_re,
        mu_table, Wp1, Wmu, Wlv):
    return pl.pallas_call(
        _vq_body,
        grid=(GRID + 1,),
        in_specs=[
            pl.BlockSpec((BN, DT), lambda i: (jnp.minimum(i, GRID - 1), 0)),
            pl.BlockSpec((BN, 1), lambda i: (jnp.minimum(i, GRID - 1), 0)),
            pl.BlockSpec((BN, DR), lambda i: (jnp.minimum(i, GRID - 1), 0)),
            pl.BlockSpec((BN, 1), lambda i: (jnp.minimum(i, GRID - 1), 0)),
            _full((KT, DT)), _full((1, KT)),
            _full((KR, DR)), _full((1, KR)),
            _full((KT, CONF)), _full((2 * DT, PH)),
            _full((PH, CONF)), _full((PH, CONF)),
        ],
        out_specs=[
            pl.BlockSpec((1, 1, BN), lambda i: (jnp.maximum(i - 1, 0), 0, 0)),
            pl.BlockSpec((1, 128), lambda i: (0, 0)),
        ],
        out_shape=[
            jax.ShapeDtypeStruct((GRID, 1, BN), jnp.int32),
            jax.ShapeDtypeStruct((1, 128), jnp.float32),
        ],
        scratch_shapes=[
            pltpu.VMEM((2 * BN, KT), jnp.float32),
            pltpu.VMEM((2 * BN, KR), jnp.float32),
            pltpu.SMEM((2,), jnp.float32),
        ],
    )(zn_tr, zsq_tr, zn_re, zsq_re, cbn_tr, cn2_tr, cbn_re, cn2_re,
      mu_table, Wp1, Wmu, Wlv)


def _sc_gather(cbn, mplv_table, idx):
    mesh = plsc.VectorSubcoreMesh(core_axis_name="c", subcore_axis_name="s")

    @functools.partial(
        pl.kernel, mesh=mesh,
        out_type=[jax.ShapeDtypeStruct((N, DT), jnp.float32),
                  jax.ShapeDtypeStruct((N, 2 * CONF), jnp.float32)],
        scratch_types=[pltpu.VMEM((RPW,), jnp.int32),
                       pltpu.VMEM((RPW, DT), jnp.float32),
                       pltpu.VMEM((RPW, 2 * CONF), jnp.float32),
                       pltpu.SemaphoreType.DMA,
                       pltpu.SemaphoreType.DMA],
    )
    def k(cbn_hbm, mplv_hbm, idx_hbm, q_out, mplv_out,
          idx_v, q_v, m_v, s1, s2):
        wid = lax.axis_index("s") * NC + lax.axis_index("c")
        base = wid * RPW
        pltpu.sync_copy(idx_hbm.at[pl.ds(base, RPW)], idx_v)
        c1 = pltpu.async_copy(cbn_hbm.at[idx_v], q_v, s1)
        c2 = pltpu.async_copy(mplv_hbm.at[idx_v], m_v, s2)
        c1.wait()
        c2.wait()
        pltpu.sync_copy(q_v, q_out.at[pl.ds(base, RPW)])
        pltpu.sync_copy(m_v, mplv_out.at[pl.ds(base, RPW)])

    return k(cbn, mplv_table, idx)


def _s3_body(z_ref, q_ref, mplv_ref, wp_ref, bp_ref, wmu_ref, bmu_ref,
             wlv_ref, blv_ref, u_ref, sc_ref, hid_s, acc):
    i = pl.program_id(0)

    @pl.when(i == 0)
    def _init():
        acc[0] = 0.0
        acc[1] = 0.0
        acc[2] = 0.0

    hid = hid_s[pl.ds(((i + 1) % 2) * BN, BN), :]
    mu = jnp.dot(hid, wmu_ref[...], preferred_element_type=jnp.float32) + bmu_ref[...]
    lv = jnp.dot(hid, wlv_ref[...], preferred_element_type=jnp.float32) + blv_ref[...]
    u_ref[...] = mu
    mplv = mplv_ref[...]
    mp = mplv[:, :CONF]
    lp = mplv[:, CONF:]
    kl_terms = lp - lv + (jnp.exp(lv) + (mu - mp) ** 2) / jnp.exp(lp) - 1.0
    acc[0] += jnp.where(i > 0, jnp.sum(kl_terms), 0.0)
    acc[1] += jnp.where(i > 0, jnp.sum(jnp.abs(mu)), 0.0)
    acc[2] += jnp.where(i > 0, jnp.sum(mp * mp), 0.0)

    xp = jnp.concatenate([z_ref[...], q_ref[...]], axis=1)
    hid_s[pl.ds((i % 2) * BN, BN), :] = jax.nn.gelu(
        jnp.dot(xp, wp_ref[...], preferred_element_type=jnp.float32) + bp_ref[...])

    @pl.when(i == GRID)
    def _fin():
        lane = lax.broadcasted_iota(jnp.int32, (1, 128), 1)
        sc_ref[...] = (jnp.where(lane == 0, acc[0], 0.0)
                       + jnp.where(lane == 1, acc[1], 0.0)
                       + jnp.where(lane == 2, acc[2], 0.0))


def _stage3(z_tr, q_tr, mplv_prior, Wp1, bp1, Wmu, bmu, Wlv, blv):
    return pl.pallas_call(
        _s3_body,
        grid=(GRID + 1,),
        in_specs=[
            pl.BlockSpec((BN, DT), lambda i: (jnp.minimum(i, GRID - 1), 0)),
            pl.BlockSpec((BN, DT), lambda i: (jnp.minimum(i, GRID - 1), 0)),
            pl.BlockSpec((BN, 2 * CONF), lambda i: (jnp.maximum(i - 1, 0), 0)),
            _full((2 * DT, PH)), _full((1, PH)),
            _full((PH, CONF)), _full((1, CONF)),
            _full((PH, CONF)), _full((1, CONF)),
        ],
        out_specs=[
            pl.BlockSpec((BN, CONF), lambda i: (jnp.maximum(i - 1, 0), 0)),
            pl.BlockSpec((1, 128), lambda i: (0, 0)),
        ],
        out_shape=[
            jax.ShapeDtypeStruct((N, CONF), jnp.float32),
            jax.ShapeDtypeStruct((1, 128), jnp.float32),
        ],
        scratch_shapes=[
            pltpu.VMEM((2 * BN, PH), jnp.float32),
            pltpu.SMEM((3,), jnp.float32),
        ],
    )(z_tr, q_tr, mplv_prior, Wp1, bp1.reshape(1, -1),
      Wmu, bmu.reshape(1, -1), Wlv, blv.reshape(1, -1))


def kernel(h, W_enc, b_enc, W_tr, b_tr, W_re, b_re, cb_tr, cb_re, mu_table,
           logvar_table, Wp1, bp1, Wmu, bmu, Wlv, blv, global_step, training):
    mplv_table = jnp.concatenate([mu_table, logvar_table], axis=1)
    z_tr, z_re = _encoder(h, W_enc, b_enc, W_tr, b_tr, W_re, b_re)

    zn_tr = z_tr / (jnp.linalg.norm(z_tr, axis=-1, keepdims=True) + 1e-6)
    zn_re = z_re / (jnp.linalg.norm(z_re, axis=-1, keepdims=True) + 1e-6)
    cbn_tr = cb_tr / (jnp.linalg.norm(cb_tr, axis=-1, keepdims=True) + 1e-6)
    cbn_re = cb_re / (jnp.linalg.norm(cb_re, axis=-1, keepdims=True) + 1e-6)
    zsq_tr = jnp.sum(zn_tr * zn_tr, axis=-1, keepdims=True)
    zsq_re = jnp.sum(zn_re * zn_re, axis=-1, keepdims=True)
    cn2_tr = jnp.sum(cbn_tr * cbn_tr, axis=-1)[None, :]
    cn2_re = jnp.sum(cbn_re * cbn_re, axis=-1)[None, :]

    idx3_tr, mse = _vq(zn_tr, zsq_tr, zn_re, zsq_re,
                       cbn_tr, cn2_tr, cbn_re, cn2_re,
                       mu_table, Wp1, Wmu, Wlv)
    idx_tr = idx3_tr.reshape(N)
    q_tr, mplv_prior = _sc_gather(cbn_tr, mplv_table, idx_tr)
    u_post, sc = _stage3(z_tr, q_tr, mplv_prior,
                         Wp1, bp1, Wmu, bmu, Wlv, blv)
    mse_tr = mse[0, 0] / (N * DT)
    mse_re = mse[0, 1] / (N * DR)
    quant = 1.25 * (mse_tr + mse_re)
    kl = 0.5 * sc[0, 0] / N
    sparsity = sc[0, 1] / (N * CONF)
    prior_reg = sc[0, 2] / (N * CONF)
    post_l2 = mse[0, 2]
    code_align = mse[0, 3] / (KT * CONF)
    conf_a = 0.1 * code_align + 0.01 * prior_reg
    conf_b = 0.1 * kl + 0.001 * post_l2 + 0.001 * sparsity
    conf = jnp.where(global_step % 3 == 0, conf_a, conf_b)
    loss = jnp.where(training, quant + conf, 0.0).astype(jnp.float32)
    return q_tr, u_post, loss
